# speedup vs baseline: 1.2672x; 1.2672x over previous
import functools

import jax
import jax.numpy as jnp
from jax import lax
from jax.experimental import pallas as pl
from jax.experimental.pallas import tpu as pltpu
from jax.experimental.pallas import tpu_sc as plsc

_N_EMBED = 128
_B = 16384
_N_SAMPLES = 3

_info = plsc.get_sparse_core_info()
_NC = _info.num_cores
_NSUB = _info.num_subcores
_NW = _NC * _NSUB

_CH = 128
_IN_CHUNKS = _B // (_NW * _CH)
_NOISE_CHUNKS = _B * _N_SAMPLES // (_NW * _CH)


def _sc_gather(iw, ow, nw, in_tab, out_tab):
    mesh = plsc.VectorSubcoreMesh(core_axis_name="c", subcore_axis_name="s")

    @functools.partial(
        pl.kernel,
        mesh=mesh,
        out_type=(
            jax.ShapeDtypeStruct((_B, _N_EMBED), jnp.float32),
            jax.ShapeDtypeStruct((_B, _N_EMBED), jnp.float32),
            jax.ShapeDtypeStruct((_B * _N_SAMPLES, _N_EMBED), jnp.float32),
        ),
        scratch_types=[
            pltpu.VMEM((_IN_CHUNKS, _CH), jnp.int32),
            pltpu.VMEM((_IN_CHUNKS, _CH), jnp.int32),
            pltpu.VMEM((_NOISE_CHUNKS, _CH), jnp.int32),
            pltpu.VMEM((2, _CH, _N_EMBED), jnp.float32),
            pltpu.SemaphoreType.DMA,
            pltpu.SemaphoreType.DMA,
        ],
    )
    def body(iw_hbm, ow_hbm, nw_hbm, in_tab_hbm, out_tab_hbm,
             o_in, o_out, o_noise,
             iw_v, ow_v, nw_v, rows_v, sem0, sem1):
        wid = lax.axis_index("s") * _NC + lax.axis_index("c")
        pltpu.sync_copy(iw_hbm.at[wid], iw_v)
        pltpu.sync_copy(ow_hbm.at[wid], ow_v)
        pltpu.sync_copy(nw_hbm.at[wid], nw_v)

        jobs = []
        for j in range(_IN_CHUNKS):
            jobs.append((in_tab_hbm, iw_v.at[j], o_in, _IN_CHUNKS, j))
        for j in range(_IN_CHUNKS):
            jobs.append((out_tab_hbm, ow_v.at[j], o_out, _IN_CHUNKS, j))
        for j in range(_NOISE_CHUNKS):
            jobs.append((out_tab_hbm, nw_v.at[j], o_noise, _NOISE_CHUNKS, j))

        sems = (sem0, sem1)
        pending = [None, None]
        for t, (tab, idxs, dst, n_chunks, j) in enumerate(jobs):
            slot = t % 2
            if pending[slot] is not None:
                copy, dst_slice = pending[slot]
                copy.wait()
                pltpu.sync_copy(rows_v.at[slot], dst_slice)
            base = (wid * n_chunks + j) * _CH
            copy = pltpu.async_copy(tab.at[idxs], rows_v.at[slot], sems[slot])
            pending[slot] = (copy, dst.at[pl.ds(base, _CH)])
        for slot in range(2):
            copy, dst_slice = pending[slot]
            copy.wait()
            pltpu.sync_copy(rows_v.at[slot], dst_slice)

    return body(iw, ow, nw, in_tab, out_tab)


def kernel(input_words, output_words, noise_words, in_embed_weight, out_embed_weight):
    iw = input_words.astype(jnp.int32).reshape(_NW, _IN_CHUNKS, _CH)
    ow = output_words.astype(jnp.int32).reshape(_NW, _IN_CHUNKS, _CH)
    nw = noise_words.astype(jnp.int32).reshape(_NW, _NOISE_CHUNKS, _CH)
    o_in, o_out, o_noise = _sc_gather(iw, ow, nw, in_embed_weight, out_embed_weight)
    return (o_in, o_out, o_noise.reshape(_B, _N_SAMPLES, _N_EMBED))


# speedup vs baseline: 1.2852x; 1.0142x over previous
import functools

import jax
import jax.numpy as jnp
from jax import lax
from jax.experimental import pallas as pl
from jax.experimental.pallas import tpu as pltpu
from jax.experimental.pallas import tpu_sc as plsc

_N_EMBED = 128
_B = 16384
_N_SAMPLES = 3

_info = plsc.get_sparse_core_info()
_NC = _info.num_cores
_NSUB = _info.num_subcores
_NW = _NC * _NSUB

_CH = 128
_NBUF = 4
_IN_CHUNKS = _B // (_NW * _CH)
_NOISE_CHUNKS = _B * _N_SAMPLES // (_NW * _CH)


def _sc_gather(iw, ow, nw, in_tab, out_tab):
    mesh = plsc.VectorSubcoreMesh(core_axis_name="c", subcore_axis_name="s")

    @functools.partial(
        pl.kernel,
        mesh=mesh,
        out_type=(
            jax.ShapeDtypeStruct((_B, _N_EMBED), jnp.float32),
            jax.ShapeDtypeStruct((_B, _N_EMBED), jnp.float32),
            jax.ShapeDtypeStruct((_B * _N_SAMPLES, _N_EMBED), jnp.float32),
        ),
        scratch_types=[
            pltpu.VMEM((_IN_CHUNKS, _CH), jnp.int32),
            pltpu.VMEM((_IN_CHUNKS, _CH), jnp.int32),
            pltpu.VMEM((_NOISE_CHUNKS, _CH), jnp.int32),
            pltpu.VMEM((_NBUF, _CH, _N_EMBED), jnp.float32),
        ]
        + [pltpu.SemaphoreType.DMA] * (2 * _NBUF),
    )
    def body(iw_hbm, ow_hbm, nw_hbm, in_tab_hbm, out_tab_hbm,
             o_in, o_out, o_noise,
             iw_v, ow_v, nw_v, rows_v, *sems):
        gsems = sems[:_NBUF]
        ssems = sems[_NBUF:]
        wid = lax.axis_index("s") * _NC + lax.axis_index("c")
        pltpu.sync_copy(iw_hbm.at[wid], iw_v)
        pltpu.sync_copy(ow_hbm.at[wid], ow_v)
        pltpu.sync_copy(nw_hbm.at[wid], nw_v)

        jobs = []
        for j in range(_IN_CHUNKS):
            jobs.append((in_tab_hbm, iw_v.at[j], o_in, _IN_CHUNKS, j))
        for j in range(_IN_CHUNKS):
            jobs.append((out_tab_hbm, ow_v.at[j], o_out, _IN_CHUNKS, j))
        for j in range(_NOISE_CHUNKS):
            jobs.append((out_tab_hbm, nw_v.at[j], o_noise, _NOISE_CHUNKS, j))
        njobs = len(jobs)

        gathers = [None] * njobs
        stores = [None] * njobs

        def issue_store(u):
            tab, idxs, dst, n_chunks, j = jobs[u]
            slot = u % _NBUF
            gathers[u].wait()
            base = (wid * n_chunks + j) * _CH
            stores[u] = pltpu.async_copy(
                rows_v.at[slot], dst.at[pl.ds(base, _CH)], ssems[slot])

        for t in range(njobs):
            tab, idxs, dst, n_chunks, j = jobs[t]
            slot = t % _NBUF
            if t >= _NBUF:
                stores[t - _NBUF].wait()
            gathers[t] = pltpu.async_copy(tab.at[idxs], rows_v.at[slot], gsems[slot])
            u = t - (_NBUF - 1)
            if u >= 0:
                issue_store(u)
        for u in range(max(0, njobs - (_NBUF - 1)), njobs):
            issue_store(u)
        for u in range(max(0, njobs - _NBUF), njobs):
            stores[u].wait()

    return body(iw, ow, nw, in_tab, out_tab)


def kernel(input_words, output_words, noise_words, in_embed_weight, out_embed_weight):
    iw = input_words.astype(jnp.int32).reshape(_NW, _IN_CHUNKS, _CH)
    ow = output_words.astype(jnp.int32).reshape(_NW, _IN_CHUNKS, _CH)
    nw = noise_words.astype(jnp.int32).reshape(_NW, _NOISE_CHUNKS, _CH)
    o_in, o_out, o_noise = _sc_gather(iw, ow, nw, in_embed_weight, out_embed_weight)
    return (o_in, o_out, o_noise.reshape(_B, _N_SAMPLES, _N_EMBED))


# speedup vs baseline: 1.6756x; 1.3038x over previous
import functools

import jax
import jax.numpy as jnp
from jax import lax
from jax.experimental import pallas as pl
from jax.experimental.pallas import tpu as pltpu
from jax.experimental.pallas import tpu_sc as plsc

_N_EMBED = 128
_B = 16384
_N_SAMPLES = 3

_info = plsc.get_sparse_core_info()
_NC = _info.num_cores
_NSUB = _info.num_subcores
_NW = _NC * _NSUB

_CH = 128
_NBUF = 3
_IN_CHUNKS = _B // (_NW * _CH)
_NOISE_CHUNKS = _B * _N_SAMPLES // (_NW * _CH)


def _sc_gather(iw, ow, nw, in_tab, out_tab):
    mesh = plsc.VectorSubcoreMesh(core_axis_name="c", subcore_axis_name="s")

    @functools.partial(
        pl.kernel,
        mesh=mesh,
        out_type=(
            jax.ShapeDtypeStruct((_B, _N_EMBED), jnp.float32),
            jax.ShapeDtypeStruct((_B, _N_EMBED), jnp.float32),
            jax.ShapeDtypeStruct((_B, _N_SAMPLES, _N_EMBED), jnp.float32),
        ),
        scratch_types=[
            pltpu.VMEM((_IN_CHUNKS, _CH), jnp.int32),
            pltpu.VMEM((_IN_CHUNKS, _CH), jnp.int32),
            pltpu.VMEM((_IN_CHUNKS, _N_SAMPLES, _CH), jnp.int32),
            pltpu.VMEM((_NBUF, _CH, _N_EMBED), jnp.float32),
            pltpu.VMEM((_NBUF, _CH, 1, _N_EMBED), jnp.float32),
        ]
        + [pltpu.SemaphoreType.DMA] * (4 * _NBUF),
    )
    def body(iw_hbm, ow_hbm, nw_hbm, in_tab_hbm, out_tab_hbm,
             o_in, o_out, o_noise,
             iw_v, ow_v, nw_v, rows2_v, rows3_v, *sems):
        wid = lax.axis_index("s") * _NC + lax.axis_index("c")
        pltpu.sync_copy(iw_hbm.at[wid], iw_v)
        pltpu.sync_copy(ow_hbm.at[wid], ow_v)
        pltpu.sync_copy(nw_hbm.at[wid], nw_v)

        jobs2 = []
        for j in range(_IN_CHUNKS):
            jobs2.append((in_tab_hbm, iw_v.at[j], o_in, j, None))
        for j in range(_IN_CHUNKS):
            jobs2.append((out_tab_hbm, ow_v.at[j], o_out, j, None))
        jobs3 = []
        for j in range(_IN_CHUNKS):
            for s in range(_N_SAMPLES):
                jobs3.append((out_tab_hbm, nw_v.at[j, s], o_noise, j, s))

        def run_ring(jobs, rows_v, gsems, ssems):
            njobs = len(jobs)
            gathers = [None] * njobs
            stores = [None] * njobs

            def issue_store(u):
                tab, idxs, dst, j, s = jobs[u]
                slot = u % _NBUF
                gathers[u].wait()
                base = (wid * _IN_CHUNKS + j) * _CH
                if s is None:
                    dst_slice = dst.at[pl.ds(base, _CH)]
                else:
                    dst_slice = dst.at[pl.ds(base, _CH), pl.ds(s, 1)]
                stores[u] = pltpu.async_copy(rows_v.at[slot], dst_slice, ssems[slot])

            for t in range(njobs):
                tab, idxs, dst, j, s = jobs[t]
                slot = t % _NBUF
                if t >= _NBUF:
                    stores[t - _NBUF].wait()
                if s is None:
                    gdst = rows_v.at[slot]
                else:
                    gdst = rows_v.at[slot, :, 0]
                gathers[t] = pltpu.async_copy(tab.at[idxs], gdst, gsems[slot])
                u = t - (_NBUF - 1)
                if u >= 0:
                    issue_store(u)
            for u in range(max(0, njobs - (_NBUF - 1)), njobs):
                issue_store(u)
            return [stores[u] for u in range(max(0, njobs - _NBUF), njobs)]

        tail = run_ring(jobs2, rows2_v, sems[:_NBUF], sems[_NBUF:2 * _NBUF])
        tail += run_ring(jobs3, rows3_v, sems[2 * _NBUF:3 * _NBUF], sems[3 * _NBUF:])
        for c in tail:
            c.wait()

    return body(iw, ow, nw, in_tab, out_tab)


def kernel(input_words, output_words, noise_words, in_embed_weight, out_embed_weight):
    iw = input_words.astype(jnp.int32).reshape(_NW, _IN_CHUNKS, _CH)
    ow = output_words.astype(jnp.int32).reshape(_NW, _IN_CHUNKS, _CH)
    nw = (noise_words.astype(jnp.int32)
          .reshape(_NW, _IN_CHUNKS, _CH, _N_SAMPLES)
          .transpose(0, 1, 3, 2))
    return _sc_gather(iw, ow, nw, in_embed_weight, out_embed_weight)


# speedup vs baseline: 2.7804x; 1.6593x over previous
import functools

import jax
import jax.numpy as jnp
from jax import lax
from jax.experimental import pallas as pl
from jax.experimental.pallas import tpu as pltpu
from jax.experimental.pallas import tpu_sc as plsc

_N_EMBED = 128
_B = 16384
_N_SAMPLES = 3

_info = plsc.get_sparse_core_info()
_NC = _info.num_cores
_NSUB = _info.num_subcores
_NL = _info.num_lanes
_NW = _NC * _NSUB

_CH = 128
_NBUF = 4
_IN_CHUNKS = _B // (_NW * _CH)
_NPW = _IN_CHUNKS * _CH * _N_SAMPLES


def _sc_gather(iw, ow, nw, in_tab, out_tab):
    mesh = plsc.VectorSubcoreMesh(core_axis_name="c", subcore_axis_name="s")

    @functools.partial(
        pl.kernel,
        mesh=mesh,
        compiler_params=pltpu.CompilerParams(needs_layout_passes=False),
        out_type=(
            jax.ShapeDtypeStruct((_B, _N_EMBED), jnp.float32),
            jax.ShapeDtypeStruct((_B, _N_EMBED), jnp.float32),
            jax.ShapeDtypeStruct((_N_SAMPLES, _B, _N_EMBED), jnp.float32),
        ),
        scratch_types=[
            pltpu.VMEM((_IN_CHUNKS, _CH), jnp.int32),
            pltpu.VMEM((_IN_CHUNKS, _CH), jnp.int32),
            pltpu.VMEM((_NPW,), jnp.int32),
            pltpu.VMEM((_NPW,), jnp.int32),
            pltpu.VMEM((_NBUF, _CH, _N_EMBED), jnp.float32),
        ]
        + [pltpu.SemaphoreType.DMA] * (2 * _NBUF),
    )
    def body(iw_hbm, ow_hbm, nw_hbm, in_tab_hbm, out_tab_hbm,
             o_in, o_out, o_noise,
             iw_v, ow_v, nw_v, nl_v, rows_v, *sems):
        gsems = sems[:_NBUF]
        ssems = sems[_NBUF:]
        wid = lax.axis_index("s") * _NC + lax.axis_index("c")
        pltpu.sync_copy(iw_hbm.at[wid], iw_v)
        pltpu.sync_copy(ow_hbm.at[wid], ow_v)
        pltpu.sync_copy(nw_hbm.at[pl.ds(wid * _NPW, _NPW)], nw_v)

        lanes = lax.iota(jnp.int32, _NL) * _N_SAMPLES
        for j in range(_IN_CHUNKS):
            for s in range(_N_SAMPLES):
                for k in range(_CH // _NL):
                    src = (j * _CH + k * _NL) * _N_SAMPLES + s
                    vals = plsc.load_gather(nw_v, [lanes + src])
                    nl_v[pl.ds(((j * _N_SAMPLES + s) * _CH + k * _NL), _NL)] = vals

        jobs = []
        for j in range(_IN_CHUNKS):
            jobs.append((in_tab_hbm, iw_v.at[j], o_in, j, None))
        for j in range(_IN_CHUNKS):
            jobs.append((out_tab_hbm, ow_v.at[j], o_out, j, None))
        for j in range(_IN_CHUNKS):
            for s in range(_N_SAMPLES):
                jobs.append((out_tab_hbm,
                             nl_v.at[pl.ds((j * _N_SAMPLES + s) * _CH, _CH)],
                             o_noise, j, s))
        njobs = len(jobs)

        gathers = [None] * njobs
        stores = [None] * njobs

        def issue_store(u):
            tab, idxs, dst, j, s = jobs[u]
            slot = u % _NBUF
            gathers[u].wait()
            base = (wid * _IN_CHUNKS + j) * _CH
            if s is None:
                dst_slice = dst.at[pl.ds(base, _CH)]
            else:
                dst_slice = dst.at[s, pl.ds(base, _CH)]
            stores[u] = pltpu.async_copy(rows_v.at[slot], dst_slice, ssems[slot])

        for t in range(njobs):
            tab, idxs, dst, j, s = jobs[t]
            slot = t % _NBUF
            if t >= _NBUF:
                stores[t - _NBUF].wait()
            gathers[t] = pltpu.async_copy(tab.at[idxs], rows_v.at[slot], gsems[slot])
            u = t - (_NBUF - 1)
            if u >= 0:
                issue_store(u)
        for u in range(max(0, njobs - (_NBUF - 1)), njobs):
            issue_store(u)
        for u in range(max(0, njobs - _NBUF), njobs):
            stores[u].wait()

    return body(iw, ow, nw, in_tab, out_tab)


def kernel(input_words, output_words, noise_words, in_embed_weight, out_embed_weight):
    iw = input_words.astype(jnp.int32).reshape(_NW, _IN_CHUNKS, _CH)
    ow = output_words.astype(jnp.int32).reshape(_NW, _IN_CHUNKS, _CH)
    nw = noise_words.astype(jnp.int32)
    o_in, o_out, o_noise = _sc_gather(iw, ow, nw, in_embed_weight, out_embed_weight)
    return (o_in, o_out, jnp.transpose(o_noise, (1, 0, 2)))


# speedup vs baseline: 2.8218x; 1.0149x over previous
import functools

import jax
import jax.numpy as jnp
from jax import lax
from jax.experimental import pallas as pl
from jax.experimental.pallas import tpu as pltpu
from jax.experimental.pallas import tpu_sc as plsc

_N_EMBED = 128
_B = 16384
_N_SAMPLES = 3

_info = plsc.get_sparse_core_info()
_NC = _info.num_cores
_NSUB = _info.num_subcores
_NL = _info.num_lanes
_NW = _NC * _NSUB

_CH = 128
_NBUF = 6
_IN_CHUNKS = _B // (_NW * _CH)
_NPW = _IN_CHUNKS * _CH * _N_SAMPLES


def _sc_gather(iw, ow, nw, in_tab, out_tab):
    mesh = plsc.VectorSubcoreMesh(core_axis_name="c", subcore_axis_name="s")

    @functools.partial(
        pl.kernel,
        mesh=mesh,
        compiler_params=pltpu.CompilerParams(needs_layout_passes=False),
        out_type=(
            jax.ShapeDtypeStruct((_B, _N_EMBED), jnp.float32),
            jax.ShapeDtypeStruct((_B, _N_EMBED), jnp.float32),
            jax.ShapeDtypeStruct((_N_SAMPLES, _B, _N_EMBED), jnp.float32),
        ),
        scratch_types=[
            pltpu.VMEM((_IN_CHUNKS, _CH), jnp.int32),
            pltpu.VMEM((_IN_CHUNKS, _CH), jnp.int32),
            pltpu.VMEM((_NPW,), jnp.int32),
            pltpu.VMEM((_NPW,), jnp.int32),
            pltpu.VMEM((_NBUF, _CH, _N_EMBED), jnp.float32),
        ]
        + [pltpu.SemaphoreType.DMA] * (2 * _NBUF),
    )
    def body(iw_hbm, ow_hbm, nw_hbm, in_tab_hbm, out_tab_hbm,
             o_in, o_out, o_noise,
             iw_v, ow_v, nw_v, nl_v, rows_v, *sems):
        gsems = sems[:_NBUF]
        ssems = sems[_NBUF:]
        wid = lax.axis_index("s") * _NC + lax.axis_index("c")
        pltpu.sync_copy(iw_hbm.at[wid], iw_v)
        pltpu.sync_copy(ow_hbm.at[wid], ow_v)
        pltpu.sync_copy(nw_hbm.at[pl.ds(wid * _NPW, _NPW)], nw_v)

        lanes = lax.iota(jnp.int32, _NL) * _N_SAMPLES
        for j in range(_IN_CHUNKS):
            for s in range(_N_SAMPLES):
                for k in range(_CH // _NL):
                    src = (j * _CH + k * _NL) * _N_SAMPLES + s
                    vals = plsc.load_gather(nw_v, [lanes + src])
                    nl_v[pl.ds(((j * _N_SAMPLES + s) * _CH + k * _NL), _NL)] = vals

        jobs = []
        for j in range(_IN_CHUNKS):
            jobs.append((in_tab_hbm, iw_v.at[j], o_in, j, None))
        for j in range(_IN_CHUNKS):
            jobs.append((out_tab_hbm, ow_v.at[j], o_out, j, None))
        for j in range(_IN_CHUNKS):
            for s in range(_N_SAMPLES):
                jobs.append((out_tab_hbm,
                             nl_v.at[pl.ds((j * _N_SAMPLES + s) * _CH, _CH)],
                             o_noise, j, s))
        njobs = len(jobs)

        gathers = [None] * njobs
        stores = [None] * njobs

        def issue_store(u):
            tab, idxs, dst, j, s = jobs[u]
            slot = u % _NBUF
            gathers[u].wait()
            base = (wid * _IN_CHUNKS + j) * _CH
            if s is None:
                dst_slice = dst.at[pl.ds(base, _CH)]
            else:
                dst_slice = dst.at[s, pl.ds(base, _CH)]
            stores[u] = pltpu.async_copy(rows_v.at[slot], dst_slice, ssems[slot])

        for t in range(njobs):
            tab, idxs, dst, j, s = jobs[t]
            slot = t % _NBUF
            if t >= _NBUF:
                stores[t - _NBUF].wait()
            gathers[t] = pltpu.async_copy(tab.at[idxs], rows_v.at[slot], gsems[slot])
            u = t - (_NBUF - 1)
            if u >= 0:
                issue_store(u)
        for u in range(max(0, njobs - (_NBUF - 1)), njobs):
            issue_store(u)
        for u in range(max(0, njobs - _NBUF), njobs):
            stores[u].wait()

    return body(iw, ow, nw, in_tab, out_tab)


def kernel(input_words, output_words, noise_words, in_embed_weight, out_embed_weight):
    iw = input_words.astype(jnp.int32).reshape(_NW, _IN_CHUNKS, _CH)
    ow = output_words.astype(jnp.int32).reshape(_NW, _IN_CHUNKS, _CH)
    nw = noise_words.astype(jnp.int32)
    o_in, o_out, o_noise = _sc_gather(iw, ow, nw, in_embed_weight, out_embed_weight)
    return (o_in, o_out, jnp.transpose(o_noise, (1, 0, 2)))
